# Initial kernel scaffold; baseline (speedup 1.0000x reference)
import jax
import jax.numpy as jnp
from jax.experimental import pallas as pl


def kernel(edge_index, edge_weight, user_emb, item_emb):
    raise NotImplementedError("write your pallas kernel here")


# speedup vs baseline: 3.2172x; 3.2172x over previous
import functools

import jax
import jax.numpy as jnp
from jax import lax
from jax.experimental import pallas as pl
from jax.experimental.pallas import tpu as pltpu
from jax.experimental.pallas import tpu_sc as plsc

N_USERS_K = 5000
N_ITEMS_K = 5000
N_TOT = N_USERS_K + N_ITEMS_K
D = 128
E_EDGES = 320000

NC = 2
NS = 16
NW = NC * NS
CHUNK = 128
CPW = -(-E_EDGES // (NW * CHUNK))
E_PAD = NW * CPW * CHUNK
ROWS_PER_TILE = 624


def _sc_round_body(emb_hbm, src_hbm, dst_hbm, w_hbm, out_hbm,
                   acc, src_idx, dst_idx, w_v, rows_v, sem):
    cid = lax.axis_index("c")
    sid = lax.axis_index("s")
    wid = sid * NC + cid

    def zero_rows(r, carry):
        for l in range(D // 16):
            rows_v[r, pl.ds(l * 16, 16)] = jnp.zeros((16,), jnp.float32)
        return carry
    lax.fori_loop(0, CHUNK, zero_rows, 0)
    base_row = sid * ROWS_PER_TILE
    for k in range(-(-ROWS_PER_TILE // CHUNK)):
        nr = min(CHUNK, ROWS_PER_TILE - k * CHUNK)
        pltpu.sync_copy(rows_v.at[pl.ds(0, nr)],
                        acc.at[pl.ds(base_row + k * CHUNK, nr)])
    tail_base = NS * ROWS_PER_TILE
    tail_rows = N_TOT - tail_base

    @pl.when(sid == 0)
    def _zero_tail():
        pltpu.sync_copy(rows_v.at[pl.ds(0, tail_rows)],
                        acc.at[pl.ds(tail_base, tail_rows)])
    plsc.subcore_barrier()

    def edge_chunk(j, carry):
        row = wid * CPW + j
        pltpu.sync_copy(src_hbm.at[row], src_idx.at[0])
        pltpu.sync_copy(dst_hbm.at[row], dst_idx.at[0])
        pltpu.sync_copy(w_hbm.at[row], w_v)
        pltpu.async_copy(emb_hbm.at[src_idx.at[0]], rows_v, sem).wait()

        def scale(g, c2):
            wvec = w_v[pl.ds(g * 16, 16)]
            for e in range(16):
                we = wvec[e]
                row = g * 16 + e
                for l in range(D // 16):
                    rows_v[row, pl.ds(l * 16, 16)] = (
                        rows_v[row, pl.ds(l * 16, 16)] * we)
            return c2
        lax.fori_loop(0, CHUNK // 16, scale, 0)

        pltpu.sync_copy(rows_v, acc.at[dst_idx.at[0]], add=True)
        return carry
    lax.fori_loop(0, CPW, edge_chunk, 0)
    plsc.subcore_barrier()

    pltpu.sync_copy(acc.at[pl.ds(base_row, ROWS_PER_TILE)],
                    out_hbm.at[pl.ds(cid * N_TOT + base_row, ROWS_PER_TILE)])

    @pl.when(sid == 0)
    def _write_tail():
        pltpu.sync_copy(acc.at[pl.ds(tail_base, tail_rows)],
                        out_hbm.at[pl.ds(cid * N_TOT + tail_base, tail_rows)])


@jax.jit
def _sc_round(emb, src2d, dst2d, w2d):
    mesh = plsc.VectorSubcoreMesh(core_axis_name="c", subcore_axis_name="s")
    return pl.kernel(
        _sc_round_body,
        out_type=jax.ShapeDtypeStruct((NC * N_TOT, D), jnp.float32),
        mesh=mesh,
        scratch_types=[
            pltpu.VMEM_SHARED((N_TOT, D), jnp.float32),
            pltpu.VMEM((1, CHUNK), jnp.int32),
            pltpu.VMEM((1, CHUNK), jnp.int32),
            pltpu.VMEM((CHUNK,), jnp.float32),
            pltpu.VMEM((CHUNK, D), jnp.float32),
            pltpu.SemaphoreType.DMA,
        ],
    )(emb, src2d, dst2d, w2d)


def _add2_body(a_ref, b_ref, o_ref):
    o_ref[...] = a_ref[...] + b_ref[...]


def _final_body(e0_ref, e1_ref, p0_ref, p1_ref, o_ref):
    o_ref[...] = (e0_ref[...] + e1_ref[...] + p0_ref[...] + p1_ref[...]) * (1.0 / 3.0)


_TC_BLK = 1000


def _tc_specs(n_in):
    spec = pl.BlockSpec((_TC_BLK, D), lambda i: (i, 0))
    return dict(
        grid=(N_TOT // _TC_BLK,),
        in_specs=[spec] * n_in,
        out_specs=spec,
        out_shape=jax.ShapeDtypeStruct((N_TOT, D), jnp.float32),
    )


@jax.jit
def _combine2(p):
    return pl.pallas_call(_add2_body, **_tc_specs(2))(p[:N_TOT], p[N_TOT:])


@jax.jit
def _final(emb0, emb1, p2):
    return pl.pallas_call(_final_body, **_tc_specs(4))(
        emb0, emb1, p2[:N_TOT], p2[N_TOT:])


def kernel(edge_index, edge_weight, user_emb, item_emb):
    emb0 = jnp.concatenate([user_emb, item_emb], axis=0)
    dst = edge_index[0]
    src = edge_index[1]
    pad = E_PAD - E_EDGES
    src2d = jnp.pad(src, (0, pad)).reshape(NW * CPW, CHUNK)
    dst2d = jnp.pad(dst, (0, pad)).reshape(NW * CPW, CHUNK)
    w2d = jnp.pad(edge_weight, (0, pad)).reshape(NW * CPW, CHUNK)

    p1 = _sc_round(emb0, src2d, dst2d, w2d)
    emb1 = _combine2(p1)
    p2 = _sc_round(emb1, src2d, dst2d, w2d)
    out = _final(emb0, emb1, p2)
    return (out[:N_USERS_K], out[N_USERS_K:])
